# Initial kernel scaffold; baseline (speedup 1.0000x reference)
import jax
import jax.numpy as jnp
from jax.experimental import pallas as pl


def kernel(x, edge_index, Wl1, Wr1, b1, Wl2, Wr2, b2, Wl3, Wr3, b3):
    raise NotImplementedError("write your pallas kernel here")


# speedup vs baseline: 2.9905x; 2.9905x over previous
import jax
import jax.numpy as jnp
from jax import lax
from jax.experimental import pallas as pl
from jax.experimental.pallas import tpu as pltpu
from jax.experimental.pallas import tpu_sc as plsc

N = 10000
E = 160000
F = 256

NC = 2
NS = 16
NW = NC * NS
EPT = E // NW
K = 40
NCH = EPT // K
ZB = 128
NZF = N // ZB
ZTAIL = N - NZF * ZB
NZ = -(-NZF // NS)

_mesh = plsc.VectorSubcoreMesh(core_axis_name="c", subcore_axis_name="s")


def _sc_agg_body(hl, srcH, dstH, zH, agg0, agg1, src_v, dst_v, rows_v, zb_v):
  c = lax.axis_index("c")
  s = lax.axis_index("s")
  wid = c * NS + s

  pltpu.sync_copy(zH, zb_v)
  for c_id, agg in ((0, agg0), (1, agg1)):
    @pl.when(c == c_id)
    def _():
      for j in range(NZ):
        blk = j * NS + s

        @pl.when(blk < NZF)
        def _():
          pltpu.sync_copy(zb_v, agg.at[pl.ds(blk * ZB, ZB)])

      @pl.when(s == 0)
      def _():
        pltpu.sync_copy(zb_v.at[pl.ds(0, ZTAIL)],
                        agg.at[pl.ds(NZF * ZB, ZTAIL)])
  plsc.subcore_barrier()

  def chunk(j, carry):
    eb = wid * EPT + j * K
    pltpu.sync_copy(srcH.at[pl.ds(eb, K)], src_v)
    pltpu.sync_copy(dstH.at[pl.ds(eb, K)], dst_v)
    pltpu.sync_copy(hl.at[src_v], rows_v)
    for c_id, agg in ((0, agg0), (1, agg1)):
      @pl.when(c == c_id)
      def _():
        pltpu.sync_copy(rows_v, agg.at[dst_v], add=True)
    return carry

  lax.fori_loop(0, NCH, chunk, 0)


_sc_agg = pl.kernel(
    _sc_agg_body,
    out_type=(jax.ShapeDtypeStruct((N, F), jnp.float32),
              jax.ShapeDtypeStruct((N, F), jnp.float32)),
    mesh=_mesh,
    scratch_types=[
        pltpu.VMEM((K,), jnp.int32),
        pltpu.VMEM((K,), jnp.int32),
        pltpu.VMEM((K, F), jnp.float32),
        pltpu.VMEM((ZB, F), jnp.float32),
    ])


CW = 256


def _sc_degree_body(dstH, zcH, onesH, cnt0, cnt1, dst_v, ones_v, zc_v):
  c = lax.axis_index("c")
  s = lax.axis_index("s")
  wid = c * NS + s

  pltpu.sync_copy(zcH, zc_v)
  pltpu.sync_copy(onesH, ones_v)
  for c_id, cnt in ((0, cnt0), (1, cnt1)):
    @pl.when(c == c_id)
    def _():
      for j in range(NZ):
        blk = j * NS + s

        @pl.when(blk < NZF)
        def _():
          pltpu.sync_copy(zc_v, cnt.at[pl.ds(blk * ZB, ZB)])

      @pl.when(s == 0)
      def _():
        pltpu.sync_copy(zc_v.at[pl.ds(0, ZTAIL)],
                        cnt.at[pl.ds(NZF * ZB, ZTAIL)])
  plsc.subcore_barrier()

  def chunk(j, carry):
    eb = wid * EPT + j * K
    pltpu.sync_copy(dstH.at[pl.ds(eb, K)], dst_v)
    for c_id, cnt in ((0, cnt0), (1, cnt1)):
      @pl.when(c == c_id)
      def _():
        pltpu.sync_copy(ones_v, cnt.at[dst_v], add=True)
    return carry

  lax.fori_loop(0, NCH, chunk, 0)


_sc_degree = pl.kernel(
    _sc_degree_body,
    out_type=(jax.ShapeDtypeStruct((N, CW), jnp.float32),
              jax.ShapeDtypeStruct((N, CW), jnp.float32)),
    mesh=_mesh,
    scratch_types=[
        pltpu.VMEM((K,), jnp.int32),
        pltpu.VMEM((K, CW), jnp.float32),
        pltpu.VMEM((ZB, CW), jnp.float32),
    ])


BN = 1000


def _tc_first_body(x_ref, wl_ref, wr_ref, b_ref, hl_ref, hrb_ref):
  h = x_ref[...]
  hl_ref[...] = jnp.dot(h, wl_ref[...], preferred_element_type=jnp.float32)
  hrb_ref[...] = (jnp.dot(h, wr_ref[...], preferred_element_type=jnp.float32)
                  + b_ref[...])


def _combine(a0_ref, a1_ref, c0_ref, c1_ref, hrb_ref):
  cnt = c0_ref[...][:, :1] + c1_ref[...][:, :1]
  inv = 1.0 / jnp.maximum(cnt, 1.0)
  return (a0_ref[...] + a1_ref[...]) * inv + hrb_ref[...]


def _tc_mid_body(a0_ref, a1_ref, c0_ref, c1_ref, hrb_ref,
                 wl_ref, wr_ref, b_ref, hl_ref, hrb_out_ref):
  h = jnp.maximum(_combine(a0_ref, a1_ref, c0_ref, c1_ref, hrb_ref), 0.0)
  hl_ref[...] = jnp.dot(h, wl_ref[...], preferred_element_type=jnp.float32)
  hrb_out_ref[...] = (jnp.dot(h, wr_ref[...],
                              preferred_element_type=jnp.float32) + b_ref[...])


def _tc_last_body(a0_ref, a1_ref, c0_ref, c1_ref, hrb_ref, out_ref):
  out_ref[...] = _combine(a0_ref, a1_ref, c0_ref, c1_ref, hrb_ref)


_row_spec = pl.BlockSpec((BN, F), lambda i: (i, 0))
_cnt_spec = pl.BlockSpec((BN, CW), lambda i: (i, 0))
_w_spec = pl.BlockSpec((F, F), lambda i: (0, 0))
_b_spec = pl.BlockSpec((1, F), lambda i: (0, 0))
_ff_out = (jax.ShapeDtypeStruct((N, F), jnp.float32),
           jax.ShapeDtypeStruct((N, F), jnp.float32))

_tc_first = pl.pallas_call(
    _tc_first_body, grid=(N // BN,),
    in_specs=[_row_spec, _w_spec, _w_spec, _b_spec],
    out_specs=(_row_spec, _row_spec), out_shape=_ff_out)

_tc_mid = pl.pallas_call(
    _tc_mid_body, grid=(N // BN,),
    in_specs=[_row_spec, _row_spec, _cnt_spec, _cnt_spec, _row_spec,
              _w_spec, _w_spec, _b_spec],
    out_specs=(_row_spec, _row_spec), out_shape=_ff_out)

_tc_last = pl.pallas_call(
    _tc_last_body, grid=(N // BN,),
    in_specs=[_row_spec, _row_spec, _cnt_spec, _cnt_spec, _row_spec],
    out_specs=_row_spec,
    out_shape=jax.ShapeDtypeStruct((N, F), jnp.float32))


def kernel(x, edge_index, Wl1, Wr1, b1, Wl2, Wr2, b2, Wl3, Wr3, b3):
  src = edge_index[0]
  dst = edge_index[1]
  zrows = jnp.zeros((ZB, F), jnp.float32)
  zcnt = jnp.zeros((ZB, CW), jnp.float32)
  ones = jnp.ones((K, CW), jnp.float32)

  c0, c1 = _sc_degree(dst, zcnt, ones)
  hl, hrb = _tc_first(x, Wl1, Wr1, b1.reshape(1, F))
  a0, a1 = _sc_agg(hl, src, dst, zrows)
  hl, hrb = _tc_mid(a0, a1, c0, c1, hrb, Wl2, Wr2, b2.reshape(1, F))
  a0, a1 = _sc_agg(hl, src, dst, zrows)
  hl, hrb = _tc_mid(a0, a1, c0, c1, hrb, Wl3, Wr3, b3.reshape(1, F))
  a0, a1 = _sc_agg(hl, src, dst, zrows)
  return _tc_last(a0, a1, c0, c1, hrb)
